# Initial kernel scaffold; baseline (speedup 1.0000x reference)
import jax
import jax.numpy as jnp
from jax.experimental import pallas as pl


def kernel(x, edge_index, game_indices, W1, b1, W2, b2, Wfc, bfc):
    raise NotImplementedError("write your pallas kernel here")


# speedup vs baseline: 4.6351x; 4.6351x over previous
import functools

import jax
import jax.numpy as jnp
from jax import lax
from jax.experimental import pallas as pl
from jax.experimental.pallas import tpu as pltpu
from jax.experimental.pallas import tpu_sc as plsc

N = 10000
E = 320000
DIN = 128
H = 256
G = 2048

NC = 2
NS = 16
L = 16

NHALF = N // NC
NOUT = N + 16
DUMMY = N
ZPT = 320

EPT = E // NS
ECH = 4000
K = 64
GB = 1024
CAP = 21504
NPAD = 10016

TPT = 312
ACCW = 328 * H
ADUM = 320

_sc_mesh = plsc.VectorSubcoreMesh(core_axis_name="c", subcore_axis_name="s")


def _prep_body(src_hbm, dst_hbm, degp_hbm, srcc_hbm, dstc_hbm, cnt_hbm,
               src_v, dst_v, degp_v, srcb_v, dstb_v, cntv_v):
    c = lax.axis_index("c")
    s = lax.axis_index("s")
    wid = c * NS + s
    lo = c * NHALF

    def zero_deg(i, _):
        degp_v[pl.ds(i * L, L)] = jnp.zeros((L,), jnp.float32)
        return 0
    lax.fori_loop(0, NPAD // L, zero_deg, 0)

    ones = jnp.ones((L,), jnp.float32)
    cnt = jnp.int32(0)
    for ci in range(EPT // ECH):
        pltpu.sync_copy(src_hbm.at[pl.ds(s * EPT + ci * ECH, ECH)], src_v)
        pltpu.sync_copy(dst_hbm.at[pl.ds(s * EPT + ci * ECH, ECH)], dst_v)

        def scan16(i, cnt):
            sv = src_v[pl.ds(i * L, L)]
            dv = dst_v[pl.ds(i * L, L)]
            plsc.addupdate_scatter(degp_v, [dv], ones)
            dloc = dv - lo
            m = (dloc >= 0) & (dloc < NHALF)
            plsc.store_compressed(srcb_v.at[pl.ds(cnt, L)], sv, mask=m)
            plsc.store_compressed(dstb_v.at[pl.ds(cnt, L)], dv, mask=m)
            return cnt + jnp.sum(m.astype(jnp.int32))
        cnt = lax.fori_loop(0, ECH // L, scan16, cnt)

    def pad16(i, _):
        srcb_v[pl.ds(cnt + i * L, L)] = jnp.zeros((L,), jnp.int32)
        dstb_v[pl.ds(cnt + i * L, L)] = jnp.full((L,), DUMMY, jnp.int32)
        return 0
    lax.fori_loop(0, GB // L, pad16, 0)

    pltpu.sync_copy(srcb_v, srcc_hbm.at[wid])
    pltpu.sync_copy(dstb_v, dstc_hbm.at[wid])
    cntv_v[...] = jnp.zeros((L,), jnp.int32) + cnt
    pltpu.sync_copy(cntv_v, cnt_hbm.at[pl.ds(wid * L, L)])

    @pl.when(c == 0)
    def _():
        pltpu.sync_copy(degp_v, degp_hbm.at[s])


_prep = pl.kernel(
    _prep_body,
    out_type=[
        jax.ShapeDtypeStruct((NS, NPAD), jnp.float32),
        jax.ShapeDtypeStruct((NC * NS, CAP), jnp.int32),
        jax.ShapeDtypeStruct((NC * NS, CAP), jnp.int32),
        jax.ShapeDtypeStruct((NC * NS * L,), jnp.int32),
    ],
    mesh=_sc_mesh,
    compiler_params=pltpu.CompilerParams(needs_layout_passes=False),
    scratch_types=[
        pltpu.VMEM((ECH,), jnp.int32),
        pltpu.VMEM((ECH,), jnp.int32),
        pltpu.VMEM((NPAD,), jnp.float32),
        pltpu.VMEM((CAP,), jnp.int32),
        pltpu.VMEM((CAP,), jnp.int32),
        pltpu.VMEM((L,), jnp.int32),
    ],
)


def _segsum_body(zp_hbm, srcc_hbm, dstc_hbm, cnt_hbm, out_hbm,
                 sidx_v, didx_v, csrc_v, cdst_v, rows_v, cnt_v, acc_v, sem):
    c = lax.axis_index("c")
    s = lax.axis_index("s")
    lo = c * NHALF + s * TPT
    mysz = jnp.where(s == NS - 1, TPT + 8, TPT)
    hi = lo + mysz

    def z16(i, _):
        acc_v[pl.ds(i * L, L)] = jnp.zeros((L,), jnp.float32)
        return 0
    lax.fori_loop(0, ACCW // L, z16, 0)
    pltpu.sync_copy(cnt_hbm, cnt_v)

    def process_chunk(qoff):
        pltpu.async_copy(zp_hbm.at[csrc_v.at[pl.ds(qoff, K)]], rows_v,
                         sem).wait()
        for grp in range(K // L):
            dv = cdst_v[pl.ds(qoff + grp * L, L)]
            for el in range(L):
                base = dv[el] * H
                for j in range(H // L):
                    val = rows_v[grp * L + el, pl.ds(j * L, L)]
                    plsc.addupdate(acc_v.at[pl.ds(base + j * L, L)], val)

    def per_worker(w2, c2):
        w = c * NS + w2
        cw = jnp.max(cnt_v[pl.ds(w * L, L)])
        nblk = (cw + (GB - 1)) // GB

        def per_block(g, c2):
            pltpu.sync_copy(srcc_hbm.at[w, pl.ds(g * GB, GB)], sidx_v)
            pltpu.sync_copy(dstc_hbm.at[w, pl.ds(g * GB, GB)], didx_v)

            def scan16(i, c2):
                sv = sidx_v[pl.ds(i * L, L)]
                dv = didx_v[pl.ds(i * L, L)]
                m = (dv >= lo) & (dv < hi)
                plsc.store_compressed(csrc_v.at[pl.ds(c2, L)], sv, mask=m)
                plsc.store_compressed(cdst_v.at[pl.ds(c2, L)], dv - lo, mask=m)
                return c2 + jnp.sum(m.astype(jnp.int32))
            c2 = lax.fori_loop(0, GB // L, scan16, c2)

            nfull = c2 // K

            def do_chunk(q, _):
                process_chunk(q * K)
                return 0
            lax.fori_loop(0, nfull, do_chunk, 0)

            base = nfull * K
            for r in range(K // L):
                csrc_v[pl.ds(r * L, L)] = csrc_v[pl.ds(base + r * L, L)]
                cdst_v[pl.ds(r * L, L)] = cdst_v[pl.ds(base + r * L, L)]
            return c2 - base
        return lax.fori_loop(0, nblk, per_block, c2)

    c2 = lax.fori_loop(0, NS, per_worker, jnp.int32(0))

    for r in range(K // L):
        csrc_v[pl.ds(c2 + r * L, L)] = jnp.zeros((L,), jnp.int32)
        cdst_v[pl.ds(c2 + r * L, L)] = jnp.full((L,), ADUM, jnp.int32)

    @pl.when(c2 > 0)
    def _():
        process_chunk(0)

    pltpu.sync_copy(acc_v.at[pl.ds(0, TPT * H)],
                    out_hbm.at[pl.ds(lo * H, TPT * H)])

    @pl.when(s == NS - 1)
    def _():
        pltpu.sync_copy(acc_v.at[pl.ds(TPT * H, 8 * H)],
                        out_hbm.at[pl.ds((lo + TPT) * H, 8 * H)])


_segsum = pl.kernel(
    _segsum_body,
    out_type=jax.ShapeDtypeStruct((NOUT * H,), jnp.float32),
    mesh=_sc_mesh,
    compiler_params=pltpu.CompilerParams(needs_layout_passes=False),
    scratch_types=[
        pltpu.VMEM((GB,), jnp.int32),
        pltpu.VMEM((GB,), jnp.int32),
        pltpu.VMEM((GB + K,), jnp.int32),
        pltpu.VMEM((GB + K,), jnp.int32),
        pltpu.VMEM((K, H), jnp.float32),
        pltpu.VMEM((NC * NS * L,), jnp.int32),
        pltpu.VMEM((ACCW,), jnp.float32),
        pltpu.SemaphoreType.DMA,
    ],
)


GPW = G // (NC * NS)


def _gather_body(logp_hbm, games_hbm, out_hbm, gidx_v, rows_v, sem):
    c = lax.axis_index("c")
    s = lax.axis_index("s")
    base = (c * NS + s) * GPW
    pltpu.sync_copy(games_hbm.at[pl.ds(base, GPW)], gidx_v)
    pltpu.async_copy(logp_hbm.at[gidx_v], rows_v, sem).wait()
    pltpu.sync_copy(rows_v, out_hbm.at[pl.ds(base, GPW)])


_gather_games = pl.kernel(
    _gather_body,
    out_type=jax.ShapeDtypeStruct((G, DIN), jnp.float32),
    mesh=_sc_mesh,
    compiler_params=pltpu.CompilerParams(needs_layout_passes=False),
    scratch_types=[
        pltpu.VMEM((GPW,), jnp.int32),
        pltpu.VMEM((GPW, DIN), jnp.float32),
        pltpu.SemaphoreType.DMA,
    ],
)


BM = 400
NBLK = N // BM


def _dis_from(degt_blk):
    deg = jnp.sum(degt_blk, axis=1)
    return lax.rsqrt(deg + 1.0)


def _tc1_body(x_ref, w1_ref, degp_ref, out_ref):
    dis = _dis_from(degp_ref[...])
    z = jnp.dot(x_ref[...], w1_ref[...], preferred_element_type=jnp.float32)
    out_ref[...] = z * dis[:, None]


def _tc1(x, w1, degp):
    return pl.pallas_call(
        _tc1_body,
        grid=(NBLK,),
        in_specs=[
            pl.BlockSpec((BM, DIN), lambda i: (i, 0)),
            pl.BlockSpec((DIN, H), lambda i: (0, 0)),
            pl.BlockSpec((BM, NS), lambda i: (i, 0)),
        ],
        out_specs=pl.BlockSpec((BM, H), lambda i: (i, 0)),
        out_shape=jax.ShapeDtypeStruct((N, H), jnp.float32),
    )(x, w1, degp)


def _tc2_body(a_ref, zp_ref, degp_ref, b_ref, w2_ref, out_ref):
    dis = _dis_from(degp_ref[...])
    h = dis[:, None] * (a_ref[...] + zp_ref[...]) + b_ref[0:1, :]
    h = jnp.maximum(h, 0.0)
    z = jnp.dot(h, w2_ref[...], preferred_element_type=jnp.float32)
    out_ref[...] = z * dis[:, None]


def _tc2(a1, zp1, degp, b1p, w2):
    return pl.pallas_call(
        _tc2_body,
        grid=(NBLK,),
        in_specs=[
            pl.BlockSpec((BM, H), lambda i: (i, 0)),
            pl.BlockSpec((BM, H), lambda i: (i, 0)),
            pl.BlockSpec((BM, NS), lambda i: (i, 0)),
            pl.BlockSpec((8, H), lambda i: (0, 0)),
            pl.BlockSpec((H, H), lambda i: (0, 0)),
        ],
        out_specs=pl.BlockSpec((BM, H), lambda i: (i, 0)),
        out_shape=jax.ShapeDtypeStruct((N, H), jnp.float32),
    )(a1, zp1, degp, b1p, w2)


def _tc3_body(a_ref, zp_ref, degp_ref, b_ref, wfc_ref, bfc_ref, out_ref):
    dis = _dis_from(degp_ref[...])
    h = dis[:, None] * (a_ref[...] + zp_ref[...]) + b_ref[0:1, :]
    h = jnp.maximum(h, 0.0)
    lg = jnp.dot(h, wfc_ref[...], preferred_element_type=jnp.float32)
    lg = lg + bfc_ref[0:1, :]
    l0 = lg[:, 0:1]
    l1 = lg[:, 1:2]
    m = jnp.maximum(l0, l1)
    lse = m + jnp.log(jnp.exp(l0 - m) + jnp.exp(l1 - m))
    out_ref[...] = lg - lse


def _tc3(a2, zp2, degp, b2p, wfcp, bfcp):
    return pl.pallas_call(
        _tc3_body,
        grid=(NBLK,),
        in_specs=[
            pl.BlockSpec((BM, H), lambda i: (i, 0)),
            pl.BlockSpec((BM, H), lambda i: (i, 0)),
            pl.BlockSpec((BM, NS), lambda i: (i, 0)),
            pl.BlockSpec((8, H), lambda i: (0, 0)),
            pl.BlockSpec((H, DIN), lambda i: (0, 0)),
            pl.BlockSpec((8, DIN), lambda i: (0, 0)),
        ],
        out_specs=pl.BlockSpec((BM, DIN), lambda i: (i, 0)),
        out_shape=jax.ShapeDtypeStruct((N, DIN), jnp.float32),
    )(a2, zp2, degp, b2p, wfcp, bfcp)


def kernel(x, edge_index, game_indices, W1, b1, W2, b2, Wfc, bfc):
    src = edge_index[0]
    dst = edge_index[1]

    degp, srcc, dstc, _cnt = _prep(src, dst)
    degt = degp.T

    b1p = jnp.pad(b1[None, :], ((0, 7), (0, 0)))
    b2p = jnp.pad(b2[None, :], ((0, 7), (0, 0)))
    wfcp = jnp.pad(Wfc, ((0, 0), (0, DIN - 2)))
    bfcp = jnp.pad(bfc[None, :], ((0, 7), (0, DIN - 2)))

    zp1 = _tc1(x, W1, degt)
    a1 = _segsum(zp1, srcc, dstc, _cnt).reshape(NOUT, H)
    zp2 = _tc2(a1, zp1, degt, b1p, W2)
    a2 = _segsum(zp2, srcc, dstc, _cnt).reshape(NOUT, H)
    logp = _tc3(a2, zp2, degt, b2p, wfcp, bfcp)
    out = _gather_games(logp, game_indices)
    return out[:, :2]


# speedup vs baseline: 5.4786x; 1.1820x over previous
import functools

import jax
import jax.numpy as jnp
from jax import lax
from jax.experimental import pallas as pl
from jax.experimental.pallas import tpu as pltpu
from jax.experimental.pallas import tpu_sc as plsc

N = 10000
E = 320000
DIN = 128
H = 256
G = 2048

NC = 2
NS = 16
L = 16

NHALF = N // NC
NOUT = N + 16
DUMMY = N
ZPT = 320

EPT = E // NS
ECH = 4000
K = 64
GB = 1024
CAP = 21504
NPAD = 10016

TPT = 312
ACCW = 328 * H
ADUM = 320

_sc_mesh = plsc.VectorSubcoreMesh(core_axis_name="c", subcore_axis_name="s")


def _prep_body(src_hbm, dst_hbm, degp_hbm, srcc_hbm, dstc_hbm, cnt_hbm,
               src_v, dst_v, degp_v, srcb_v, dstb_v, cntv_v):
    c = lax.axis_index("c")
    s = lax.axis_index("s")
    wid = c * NS + s
    lo = c * NHALF

    def zero_deg(i, _):
        degp_v[pl.ds(i * L, L)] = jnp.zeros((L,), jnp.float32)
        return 0
    lax.fori_loop(0, NPAD // L, zero_deg, 0)

    ones = jnp.ones((L,), jnp.float32)
    cnt = jnp.int32(0)
    for ci in range(EPT // ECH):
        pltpu.sync_copy(src_hbm.at[pl.ds(s * EPT + ci * ECH, ECH)], src_v)
        pltpu.sync_copy(dst_hbm.at[pl.ds(s * EPT + ci * ECH, ECH)], dst_v)

        def scan16(i, cnt):
            sv = src_v[pl.ds(i * L, L)]
            dv = dst_v[pl.ds(i * L, L)]
            plsc.addupdate_scatter(degp_v, [dv], ones)
            dloc = dv - lo
            m = (dloc >= 0) & (dloc < NHALF)
            plsc.store_compressed(srcb_v.at[pl.ds(cnt, L)], sv, mask=m)
            plsc.store_compressed(dstb_v.at[pl.ds(cnt, L)], dv, mask=m)
            return cnt + jnp.sum(m.astype(jnp.int32))
        cnt = lax.fori_loop(0, ECH // L, scan16, cnt)

    def pad16(i, _):
        srcb_v[pl.ds(cnt + i * L, L)] = jnp.zeros((L,), jnp.int32)
        dstb_v[pl.ds(cnt + i * L, L)] = jnp.full((L,), DUMMY, jnp.int32)
        return 0
    lax.fori_loop(0, GB // L, pad16, 0)

    pltpu.sync_copy(srcb_v, srcc_hbm.at[wid])
    pltpu.sync_copy(dstb_v, dstc_hbm.at[wid])
    cntv_v[...] = jnp.zeros((L,), jnp.int32) + cnt
    pltpu.sync_copy(cntv_v, cnt_hbm.at[pl.ds(wid * L, L)])

    @pl.when(c == 0)
    def _():
        pltpu.sync_copy(degp_v, degp_hbm.at[s])


_prep = pl.kernel(
    _prep_body,
    out_type=[
        jax.ShapeDtypeStruct((NS, NPAD), jnp.float32),
        jax.ShapeDtypeStruct((NC * NS, CAP), jnp.int32),
        jax.ShapeDtypeStruct((NC * NS, CAP), jnp.int32),
        jax.ShapeDtypeStruct((NC * NS * L,), jnp.int32),
    ],
    mesh=_sc_mesh,
    compiler_params=pltpu.CompilerParams(needs_layout_passes=False),
    scratch_types=[
        pltpu.VMEM((ECH,), jnp.int32),
        pltpu.VMEM((ECH,), jnp.int32),
        pltpu.VMEM((NPAD,), jnp.float32),
        pltpu.VMEM((CAP,), jnp.int32),
        pltpu.VMEM((CAP,), jnp.int32),
        pltpu.VMEM((L,), jnp.int32),
    ],
)


def _segsum_body(zp_hbm, srcc_hbm, dstc_hbm, cnt_hbm, out_hbm,
                 sidx_v, didx_v, csrc_v, cdst_v, rows_v, cnt_v, acc_v, sem):
    c = lax.axis_index("c")
    s = lax.axis_index("s")
    lo = c * NHALF + s * TPT
    mysz = jnp.where(s == NS - 1, TPT + 8, TPT)
    hi = lo + mysz

    def z16(i, _):
        acc_v[pl.ds(i * L, L)] = jnp.zeros((L,), jnp.float32)
        return 0
    lax.fori_loop(0, ACCW // L, z16, 0)
    pltpu.sync_copy(cnt_hbm, cnt_v)

    def process_chunk(qoff):
        pltpu.async_copy(zp_hbm.at[csrc_v.at[pl.ds(qoff, K)]], rows_v,
                         sem).wait()
        for grp in range(K // L):
            dv = cdst_v[pl.ds(qoff + grp * L, L)]
            bases = [dv[el] * H for el in range(L)]
            for el in range(L):
                vals = [rows_v[grp * L + el, pl.ds(j * L, L)]
                        for j in range(H // L)]
                for j in range(H // L):
                    plsc.addupdate(acc_v.at[pl.ds(bases[el] + j * L, L)],
                                   vals[j])

    def per_worker(w2, c2):
        w = c * NS + w2
        cw = jnp.max(cnt_v[pl.ds(w * L, L)])
        nblk = (cw + (GB - 1)) // GB

        def per_block(g, c2):
            pltpu.sync_copy(srcc_hbm.at[w, pl.ds(g * GB, GB)], sidx_v)
            pltpu.sync_copy(dstc_hbm.at[w, pl.ds(g * GB, GB)], didx_v)

            def scan16(i, c2):
                sv = sidx_v[pl.ds(i * L, L)]
                dv = didx_v[pl.ds(i * L, L)]
                m = (dv >= lo) & (dv < hi)
                plsc.store_compressed(csrc_v.at[pl.ds(c2, L)], sv, mask=m)
                plsc.store_compressed(cdst_v.at[pl.ds(c2, L)], dv - lo, mask=m)
                return c2 + jnp.sum(m.astype(jnp.int32))
            c2 = lax.fori_loop(0, GB // L, scan16, c2)

            nfull = c2 // K

            def do_chunk(q, _):
                process_chunk(q * K)
                return 0
            lax.fori_loop(0, nfull, do_chunk, 0)

            base = nfull * K
            for r in range(K // L):
                csrc_v[pl.ds(r * L, L)] = csrc_v[pl.ds(base + r * L, L)]
                cdst_v[pl.ds(r * L, L)] = cdst_v[pl.ds(base + r * L, L)]
            return c2 - base
        return lax.fori_loop(0, nblk, per_block, c2)

    c2 = lax.fori_loop(0, NS, per_worker, jnp.int32(0))

    for r in range(K // L):
        csrc_v[pl.ds(c2 + r * L, L)] = jnp.zeros((L,), jnp.int32)
        cdst_v[pl.ds(c2 + r * L, L)] = jnp.full((L,), ADUM, jnp.int32)

    @pl.when(c2 > 0)
    def _():
        process_chunk(0)

    pltpu.sync_copy(acc_v.at[pl.ds(0, TPT * H)],
                    out_hbm.at[pl.ds(lo * H, TPT * H)])

    @pl.when(s == NS - 1)
    def _():
        pltpu.sync_copy(acc_v.at[pl.ds(TPT * H, 8 * H)],
                        out_hbm.at[pl.ds((lo + TPT) * H, 8 * H)])


_segsum = pl.kernel(
    _segsum_body,
    out_type=jax.ShapeDtypeStruct((NOUT * H,), jnp.float32),
    mesh=_sc_mesh,
    compiler_params=pltpu.CompilerParams(needs_layout_passes=False),
    scratch_types=[
        pltpu.VMEM((GB,), jnp.int32),
        pltpu.VMEM((GB,), jnp.int32),
        pltpu.VMEM((GB + K,), jnp.int32),
        pltpu.VMEM((GB + K,), jnp.int32),
        pltpu.VMEM((K, H), jnp.float32),
        pltpu.VMEM((NC * NS * L,), jnp.int32),
        pltpu.VMEM((ACCW,), jnp.float32),
        pltpu.SemaphoreType.DMA,
    ],
)


GPW = G // (NC * NS)


def _gather_body(logp_hbm, games_hbm, out_hbm, gidx_v, rows_v, sem):
    c = lax.axis_index("c")
    s = lax.axis_index("s")
    base = (c * NS + s) * GPW
    pltpu.sync_copy(games_hbm.at[pl.ds(base, GPW)], gidx_v)
    pltpu.async_copy(logp_hbm.at[gidx_v], rows_v, sem).wait()
    pltpu.sync_copy(rows_v, out_hbm.at[pl.ds(base, GPW)])


_gather_games = pl.kernel(
    _gather_body,
    out_type=jax.ShapeDtypeStruct((G, DIN), jnp.float32),
    mesh=_sc_mesh,
    compiler_params=pltpu.CompilerParams(needs_layout_passes=False),
    scratch_types=[
        pltpu.VMEM((GPW,), jnp.int32),
        pltpu.VMEM((GPW, DIN), jnp.float32),
        pltpu.SemaphoreType.DMA,
    ],
)


BM = 400
NBLK = N // BM


def _dis_from(degt_blk):
    deg = jnp.sum(degt_blk, axis=1)
    return lax.rsqrt(deg + 1.0)


def _tc1_body(x_ref, w1_ref, degp_ref, out_ref):
    dis = _dis_from(degp_ref[...])
    z = jnp.dot(x_ref[...], w1_ref[...], preferred_element_type=jnp.float32)
    out_ref[...] = z * dis[:, None]


def _tc1(x, w1, degp):
    return pl.pallas_call(
        _tc1_body,
        grid=(NBLK,),
        in_specs=[
            pl.BlockSpec((BM, DIN), lambda i: (i, 0)),
            pl.BlockSpec((DIN, H), lambda i: (0, 0)),
            pl.BlockSpec((BM, NS), lambda i: (i, 0)),
        ],
        out_specs=pl.BlockSpec((BM, H), lambda i: (i, 0)),
        out_shape=jax.ShapeDtypeStruct((N, H), jnp.float32),
    )(x, w1, degp)


def _tc2_body(a_ref, zp_ref, degp_ref, b_ref, w2_ref, out_ref):
    dis = _dis_from(degp_ref[...])
    h = dis[:, None] * (a_ref[...] + zp_ref[...]) + b_ref[0:1, :]
    h = jnp.maximum(h, 0.0)
    z = jnp.dot(h, w2_ref[...], preferred_element_type=jnp.float32)
    out_ref[...] = z * dis[:, None]


def _tc2(a1, zp1, degp, b1p, w2):
    return pl.pallas_call(
        _tc2_body,
        grid=(NBLK,),
        in_specs=[
            pl.BlockSpec((BM, H), lambda i: (i, 0)),
            pl.BlockSpec((BM, H), lambda i: (i, 0)),
            pl.BlockSpec((BM, NS), lambda i: (i, 0)),
            pl.BlockSpec((8, H), lambda i: (0, 0)),
            pl.BlockSpec((H, H), lambda i: (0, 0)),
        ],
        out_specs=pl.BlockSpec((BM, H), lambda i: (i, 0)),
        out_shape=jax.ShapeDtypeStruct((N, H), jnp.float32),
    )(a1, zp1, degp, b1p, w2)


def _tc3_body(a_ref, zp_ref, degp_ref, b_ref, wfc_ref, bfc_ref, out_ref):
    dis = _dis_from(degp_ref[...])
    h = dis[:, None] * (a_ref[...] + zp_ref[...]) + b_ref[0:1, :]
    h = jnp.maximum(h, 0.0)
    lg = jnp.dot(h, wfc_ref[...], preferred_element_type=jnp.float32)
    lg = lg + bfc_ref[0:1, :]
    l0 = lg[:, 0:1]
    l1 = lg[:, 1:2]
    m = jnp.maximum(l0, l1)
    lse = m + jnp.log(jnp.exp(l0 - m) + jnp.exp(l1 - m))
    out_ref[...] = lg - lse


def _tc3(a2, zp2, degp, b2p, wfcp, bfcp):
    return pl.pallas_call(
        _tc3_body,
        grid=(NBLK,),
        in_specs=[
            pl.BlockSpec((BM, H), lambda i: (i, 0)),
            pl.BlockSpec((BM, H), lambda i: (i, 0)),
            pl.BlockSpec((BM, NS), lambda i: (i, 0)),
            pl.BlockSpec((8, H), lambda i: (0, 0)),
            pl.BlockSpec((H, DIN), lambda i: (0, 0)),
            pl.BlockSpec((8, DIN), lambda i: (0, 0)),
        ],
        out_specs=pl.BlockSpec((BM, DIN), lambda i: (i, 0)),
        out_shape=jax.ShapeDtypeStruct((N, DIN), jnp.float32),
    )(a2, zp2, degp, b2p, wfcp, bfcp)


def kernel(x, edge_index, game_indices, W1, b1, W2, b2, Wfc, bfc):
    src = edge_index[0]
    dst = edge_index[1]

    degp, srcc, dstc, _cnt = _prep(src, dst)
    degt = degp.T

    b1p = jnp.pad(b1[None, :], ((0, 7), (0, 0)))
    b2p = jnp.pad(b2[None, :], ((0, 7), (0, 0)))
    wfcp = jnp.pad(Wfc, ((0, 0), (0, DIN - 2)))
    bfcp = jnp.pad(bfc[None, :], ((0, 7), (0, DIN - 2)))

    zp1 = _tc1(x, W1, degt)
    a1 = _segsum(zp1, srcc, dstc, _cnt).reshape(NOUT, H)
    zp2 = _tc2(a1, zp1, degt, b1p, W2)
    a2 = _segsum(zp2, srcc, dstc, _cnt).reshape(NOUT, H)
    logp = _tc3(a2, zp2, degt, b2p, wfcp, bfcp)
    out = _gather_games(logp, game_indices)
    return out[:, :2]
